# baseline (speedup 1.0000x reference)
import functools

import jax
import jax.numpy as jnp
from jax import lax
from jax.experimental import pallas as pl
from jax.experimental.pallas import tpu as pltpu
from jax.experimental.pallas import tpu_sc as plsc

N_NODES_ = 10000
D_ = 128
N_POOL_ = 5000
N_EDGES_ = 320000
NW_ = 32

PP_ = 160
PB_ = 4840
EW_ = N_EDGES_ // NW_
CR_ = 320
CB_ = N_NODES_ - CR_
CH_ = 160


def _unpool_body(feat_hbm, pool_hbm, edge_hbm,
                 outf_hbm, src_hbm, dst_hbm,
                 idx_v, rows_v, newv_v, ebuf_v, c0_v, c1_v, cbuf_v,
                 gsem, esem, nsem):
    wid = lax.axis_index("s") * 2 + lax.axis_index("c")

    base_p = jnp.minimum(wid * PP_, PB_)
    base_e = wid * EW_
    base_c = jnp.minimum(wid * CR_, CB_)

    pltpu.sync_copy(pool_hbm.at[pl.ds(2 * base_p, 2 * PP_)], idx_v)
    gcopy = pltpu.async_copy(feat_hbm.at[idx_v], rows_v, gsem)
    ecopy = pltpu.async_copy(edge_hbm.at[pl.ds(2 * base_e, 2 * EW_)], ebuf_v, esem)

    for off in (0, CH_):
        pltpu.sync_copy(feat_hbm.at[pl.ds(base_c + off, CH_)], cbuf_v)
        pltpu.sync_copy(cbuf_v, outf_hbm.at[pl.ds(base_c + off, CH_)])

    gcopy.wait()

    def avg_row(j, carry):
        for d in range(D_ // 16):
            a = rows_v[2 * j, pl.ds(16 * d, 16)]
            b = rows_v[2 * j + 1, pl.ds(16 * d, 16)]
            newv_v[j, pl.ds(16 * d, 16)] = 0.5 * (a + b)
        return carry

    lax.fori_loop(0, PP_, avg_row, 0, unroll=2)
    ncopy = pltpu.async_copy(newv_v, outf_hbm.at[pl.ds(N_NODES_ + base_p, PP_)], nsem)

    ecopy.wait()
    lanes = lax.iota(jnp.int32, 16)

    def deint(i, carry):
        iv = 32 * i + 2 * lanes
        c0_v[pl.ds(16 * i, 16)] = plsc.load_gather(ebuf_v, [iv])
        c1_v[pl.ds(16 * i, 16)] = plsc.load_gather(ebuf_v, [iv + 1])
        return carry

    lax.fori_loop(0, EW_ // 16, deint, 0, unroll=4)

    pltpu.sync_copy(c0_v, src_hbm.at[pl.ds(base_e, EW_)])
    pltpu.sync_copy(c1_v, src_hbm.at[pl.ds(N_EDGES_ + base_e, EW_)])
    pltpu.sync_copy(c1_v, dst_hbm.at[pl.ds(base_e, EW_)])
    pltpu.sync_copy(c0_v, dst_hbm.at[pl.ds(N_EDGES_ + base_e, EW_)])
    ncopy.wait()


_unpool_sc = functools.partial(
    pl.kernel,
    out_type=[
        jax.ShapeDtypeStruct((N_NODES_ + N_POOL_, D_), jnp.float32),
        jax.ShapeDtypeStruct((2 * N_EDGES_,), jnp.int32),
        jax.ShapeDtypeStruct((2 * N_EDGES_,), jnp.int32),
    ],
    mesh=plsc.VectorSubcoreMesh(core_axis_name="c", subcore_axis_name="s"),
    compiler_params=pltpu.CompilerParams(needs_layout_passes=False),
    scratch_types=[
        pltpu.VMEM((2 * PP_,), jnp.int32),
        pltpu.VMEM((2 * PP_, D_), jnp.float32),
        pltpu.VMEM((PP_, D_), jnp.float32),
        pltpu.VMEM((2 * EW_,), jnp.int32),
        pltpu.VMEM((EW_,), jnp.int32),
        pltpu.VMEM((EW_,), jnp.int32),
        pltpu.VMEM((CH_, D_), jnp.float32),
        pltpu.SemaphoreType.DMA,
        pltpu.SemaphoreType.DMA,
        pltpu.SemaphoreType.DMA,
    ],
)(_unpool_body)


@jax.jit
def kernel(feat, pool_idx_, edge_idx_):
    pool_flat = pool_idx_.astype(jnp.int32).reshape(2 * N_POOL_)
    edge_flat = edge_idx_.astype(jnp.int32).reshape(2 * N_EDGES_)
    feat_out, src_all, dst_all = _unpool_sc(feat, pool_flat, edge_flat)
    return feat_out, src_all, dst_all
